# Initial kernel scaffold; baseline (speedup 1.0000x reference)
import jax
import jax.numpy as jnp
from jax.experimental import pallas as pl


def kernel(node_features, adjacency_matrix, edge_types, W1, b1, W2, b2, Ws1, bs1, Ws2, bs2, Wa1, ba1, Wa2, ba2):
    raise NotImplementedError("write your pallas kernel here")


# speedup vs baseline: 77.6880x; 77.6880x over previous
import jax
import jax.numpy as jnp
from jax.experimental import pallas as pl


_N = 1024
_D = 512
_BIG = 1 << 30


def _fused_tc_kernel(adj_ref, nf_ref, w1_ref, b1_ref, w2_ref, b2_ref,
                     wa1_ref, ba1_ref, wa2_ref, ba2_ref, out_ref):
    adj = adj_ref[...]
    mask = adj > 0.0
    cnt = jnp.sum(mask.astype(jnp.int32))

    pos = (jax.lax.broadcasted_iota(jnp.int32, (_N, _N), 0) * _N
           + jax.lax.broadcasted_iota(jnp.int32, (_N, _N), 1))
    p = jnp.where(mask, pos, _BIG)
    first = jnp.min(p)
    second = jnp.min(jnp.where(p == first, _BIG, p))

    idx0 = jnp.where(cnt >= 1, first, 0)
    idx1 = jnp.where(cnt >= 2, second, 0)
    src0 = idx0 // _N
    dst0 = idx0 % _N
    src1 = idx1 // _N
    dst1 = idx1 % _N

    col = jax.lax.broadcasted_iota(jnp.int32, (4, _N), 1)
    sel = jnp.concatenate(
        [src0.reshape(1, 1), dst0.reshape(1, 1),
         src1.reshape(1, 1), dst1.reshape(1, 1)], axis=0)
    s_onehot = (col == sel).astype(jnp.float32)
    g = jnp.dot(s_onehot, nf_ref[...], preferred_element_type=jnp.float32)

    x = jnp.concatenate(
        [jnp.concatenate([g[0:1, :], g[1:2, :]], axis=1),
         jnp.concatenate([g[2:3, :], g[3:4, :]], axis=1)], axis=0)

    hp = jax.lax.dot_general(
        x, w1_ref[...], (((1,), (0,)), ((), ())),
        preferred_element_type=jnp.float32) + b1_ref[...]
    step = jax.lax.dot_general(
        jnp.maximum(hp, 0.0), w2_ref[...], (((1,), (0,)), ((), ())),
        preferred_element_type=jnp.float32) + b2_ref[...]

    h0 = jax.lax.dot_general(
        step[0:1, :], wa1_ref[0:_D, :], (((1,), (0,)), ((), ())),
        preferred_element_type=jnp.float32)
    h1 = jax.lax.dot_general(
        step[1:2, :], wa1_ref[_D:2 * _D, :], (((1,), (0,)), ((), ())),
        preferred_element_type=jnp.float32)
    h = jnp.maximum(h0 + h1 + ba1_ref[...], 0.0)
    agg = jax.lax.dot_general(
        h, wa2_ref[...], (((1,), (0,)), ((), ())),
        preferred_element_type=jnp.float32) + ba2_ref[...]

    addv = jnp.where(cnt > 0, agg, jnp.zeros_like(agg))
    out_ref[...] = nf_ref[...] + addv


def kernel(node_features, adjacency_matrix, edge_types, W1, b1, W2, b2,
           Ws1, bs1, Ws2, bs2, Wa1, ba1, Wa2, ba2):
    del edge_types, Ws1, bs1, Ws2, bs2
    return pl.pallas_call(
        _fused_tc_kernel,
        out_shape=jax.ShapeDtypeStruct((_N, _D), jnp.float32),
    )(adjacency_matrix, node_features, W1, b1.reshape(1, _D), W2,
      b2.reshape(1, _D), Wa1, ba1.reshape(1, _D), Wa2, ba2.reshape(1, _D))
